# baseline (speedup 1.0000x reference)
import functools
import jax
import jax.numpy as jnp
from jax import lax
from jax.experimental import pallas as pl
from jax.experimental.pallas import tpu as pltpu
from jax.experimental.pallas import tpu_sc as plsc

F = 26
D = 4
VF = 4000
B = 4096
NPAIRS = F * (F - 1) // 2
L = 16
NB = B // L
NC = 2
NS = 16
NW = NC * NS
U = 8
SW = VF * 2
RW = SW * F // 128
TW = 72

_START = [0]
for _i in range(F):
    _START.append(_START[-1] + (F - 1 - _i))


def _decode_pair(k):
    i = jnp.int32(0)
    for m in range(1, F - 1):
        i = i + jnp.where(k >= _START[m], 1, 0).astype(jnp.int32)
    start_i = i * (F - 1) - ((i * (i - 1)) >> 1)
    j = k - start_i + i + 1
    return i, j


def _pair_params(wid, p):
    if p < 10:
        k = jnp.int32(p * NW) + wid
        scale = jnp.float32(1.0)
    else:
        k = jnp.int32(320 - 27) + wid
        k = jnp.minimum(k, NPAIRS - 1)
        scale = jnp.where(wid >= 27, 1.0, 0.0).astype(jnp.float32)
    i, j = _decode_pair(k)
    w1 = i * SW
    w2 = j * SW
    a1 = pl.multiple_of((w1 >> 7) & ~7, 8)
    a2 = pl.multiple_of((w2 >> 7) & ~7, 8)
    return i, j, a1, a2, w1 - (a1 << 7), w2 - (a2 << 7), scale


def _k1_body(wcr, wlin, xt, part,
             tblA1, tblA2, tblB1, tblB2, xiA, xjA, xiB, xjB,
             lin_v, xlin_v, acc_v, semA, semB):
    cid = lax.axis_index("c")
    sid = lax.axis_index("s")
    wid = sid * NC + cid

    bufs = [(tblA1, tblA2, xiA, xjA, semA), (tblB1, tblB2, xiB, xjB, semB)]

    def start(p):
        i, j, a1, a2, _, _, _ = _pair_params(wid, p)
        t1, t2, xi, xj, sem = bufs[p % 2]
        return (pltpu.async_copy(wcr.at[j, pl.ds(a1, TW)], t1, sem),
                pltpu.async_copy(wcr.at[i, pl.ds(a2, TW)], t2, sem),
                pltpu.async_copy(xt.at[i], xi, sem),
                pltpu.async_copy(xt.at[j], xj, sem))

    descs = start(0)

    lin_valid = wid < F
    f = jnp.minimum(wid, F - 1)
    lin_splat = jnp.full((L,), jnp.where(lin_valid, 1.0, 0.0), jnp.float32)
    pltpu.sync_copy(wlin.at[f], lin_v)
    pltpu.sync_copy(xt.at[f], xlin_v)

    def lb(s0, c):
        base = s0 * (U * L)
        for u in range(U):
            sl = pl.ds(base + u * L, L)
            idx = xlin_v[sl]
            vals = plsc.load_gather(lin_v, [idx])
            acc_v[sl] = vals * lin_splat
        return c
    lax.fori_loop(0, NB // U, lb, 0)

    for p in range(11):
        t1, t2, xi, xj, _ = bufs[p % 2]
        nxt = start(p + 1) if p + 1 < 11 else None
        for dsc in descs:
            dsc.wait()
        descs = nxt
        _, _, _, _, ro1, ro2, scale = _pair_params(wid, p)
        roff1 = jnp.full((L,), ro1, jnp.int32)
        roff2 = jnp.full((L,), ro2, jnp.int32)
        scale_splat = jnp.full((L,), scale, jnp.float32)

        himask = jnp.full((L,), -65536, jnp.int32)

        def pb(s0, c):
            base = s0 * (U * L)
            for u in range(U):
                sl = pl.ds(base + u * L, L)
                wi = xi[sl] * 2 + roff1
                wj = xj[sl] * 2 + roff2
                tot = None
                for dp in range(2):
                    fi = wi + dp
                    fj = wj + dp
                    e1 = plsc.load_gather(t1, [fi >> 7, fi & 127])
                    e2 = plsc.load_gather(t2, [fj >> 7, fj & 127])
                    p = plsc.bitcast(plsc.bitcast(e1, jnp.bfloat16)
                                     * plsc.bitcast(e2, jnp.bfloat16),
                                     jnp.int32)
                    lo = plsc.bitcast(p << 16, jnp.float32)
                    hi = plsc.bitcast(p & himask, jnp.float32)
                    s2 = lo + hi
                    tot = s2 if tot is None else tot + s2
                acc_v[sl] = acc_v[sl] + tot * scale_splat
            return c
        lax.fori_loop(0, NB // U, pb, 0)

    pltpu.sync_copy(acc_v, part.at[wid])


def _k2_body(part, bias16, out, buf_v, bias_v, out_v):
    cid = lax.axis_index("c")
    sid = lax.axis_index("s")
    wid = sid * NC + cid
    bw = B // NW
    cbase = wid * bw

    pltpu.sync_copy(part.at[:, pl.ds(cbase, bw)], buf_v)
    pltpu.sync_copy(bias16, bias_v)
    bv = bias_v[...]

    for v in range(bw // L):
        acc = jnp.zeros((L,), jnp.float32)
        for r in range(NW):
            acc = acc + buf_v[r, pl.ds(v * L, L)]
        z = acc + bv
        res = 1.0 / (1.0 + jnp.exp(-z))
        out_v[pl.ds(v * L, L)] = res

    pltpu.sync_copy(out_v, out.at[pl.ds(cbase, bw)])


_mesh = plsc.VectorSubcoreMesh(core_axis_name="c", subcore_axis_name="s")

_cparams = pltpu.CompilerParams(needs_layout_passes=False,
                               use_tc_tiling_on_sc=True)

_k1 = pl.kernel(
    _k1_body,
    out_type=jax.ShapeDtypeStruct((NW, B), jnp.float32),
    mesh=_mesh,
    compiler_params=_cparams,
    scratch_types=[
        pltpu.VMEM((TW, 128), jnp.int32),
        pltpu.VMEM((TW, 128), jnp.int32),
        pltpu.VMEM((TW, 128), jnp.int32),
        pltpu.VMEM((TW, 128), jnp.int32),
        pltpu.VMEM((B,), jnp.int32),
        pltpu.VMEM((B,), jnp.int32),
        pltpu.VMEM((B,), jnp.int32),
        pltpu.VMEM((B,), jnp.int32),
        pltpu.VMEM((VF,), jnp.float32),
        pltpu.VMEM((B,), jnp.int32),
        pltpu.VMEM((B,), jnp.float32),
        pltpu.SemaphoreType.DMA,
        pltpu.SemaphoreType.DMA,
    ],
)

_k2 = pl.kernel(
    _k2_body,
    out_type=jax.ShapeDtypeStruct((B,), jnp.float32),
    mesh=_mesh,
    compiler_params=_cparams,
    scratch_types=[
        pltpu.VMEM((NW, B // NW), jnp.float32),
        pltpu.VMEM((L,), jnp.float32),
        pltpu.VMEM((B // NW,), jnp.float32),
    ],
)


@jax.jit
def kernel(x, W_lin, W_cross, bias):
    wcb = W_cross.astype(jnp.bfloat16).reshape(F, F * VF, 2, 2)
    wcr = jax.lax.bitcast_convert_type(wcb, jnp.int32).reshape(F, RW, 128)
    wlin = W_lin.reshape(F, VF)
    xt = x.T
    bias16 = jnp.broadcast_to(bias.astype(jnp.float32), (L,))
    part = _k1(wcr, wlin, xt)
    return _k2(part, bias16)


# speedup vs baseline: 1.1225x; 1.1225x over previous
import functools
import jax
import jax.numpy as jnp
from jax import lax
from jax.experimental import pallas as pl
from jax.experimental.pallas import tpu as pltpu
from jax.experimental.pallas import tpu_sc as plsc

F = 26
D = 4
VF = 4000
B = 4096
NPAIRS = F * (F - 1) // 2
L = 16
NB = B // L
NC = 2
NS = 16
NW = NC * NS
U = 8
TR = VF * D // 128
SR = TR * F
TW = 136

_START = [0]
for _i in range(F):
    _START.append(_START[-1] + (F - 1 - _i))


def _decode_pair(k):
    i = jnp.int32(0)
    for m in range(1, F - 1):
        i = i + jnp.where(k >= _START[m], 1, 0).astype(jnp.int32)
    start_i = i * (F - 1) - ((i * (i - 1)) >> 1)
    j = k - start_i + i + 1
    return i, j


def _pair_params(wid, p):
    if p < 10:
        k = jnp.int32(p * NW) + wid
        scale = jnp.float32(1.0)
    else:
        k = jnp.int32(320 - 27) + wid
        k = jnp.minimum(k, NPAIRS - 1)
        scale = jnp.where(wid >= 27, 1.0, 0.0).astype(jnp.float32)
    i, j = _decode_pair(k)
    r1 = i * TR
    r2 = j * TR
    a1 = pl.multiple_of(r1 & ~7, 8)
    a2 = pl.multiple_of(r2 & ~7, 8)
    return i, j, a1, a2, r1 - a1, r2 - a2, scale


def _k1_body(wcr, wlin, xt, part,
             tblA1, tblA2, tblB1, tblB2, xiA, xjA, xiB, xjB,
             lin_v, xlin_v, acc_v, semA, semB):
    cid = lax.axis_index("c")
    sid = lax.axis_index("s")
    wid = sid * NC + cid

    bufs = [(tblA1, tblA2, xiA, xjA, semA), (tblB1, tblB2, xiB, xjB, semB)]

    def start(p):
        i, j, a1, a2, _, _, _ = _pair_params(wid, p)
        t1, t2, xi, xj, sem = bufs[p % 2]
        return (pltpu.async_copy(wcr.at[j, pl.ds(a1, TW)], t1, sem),
                pltpu.async_copy(wcr.at[i, pl.ds(a2, TW)], t2, sem),
                pltpu.async_copy(xt.at[i], xi, sem),
                pltpu.async_copy(xt.at[j], xj, sem))

    descs = start(0)

    lin_valid = wid < F
    f = jnp.minimum(wid, F - 1)
    lin_splat = jnp.full((L,), jnp.where(lin_valid, 1.0, 0.0), jnp.float32)
    pltpu.sync_copy(wlin.at[f], lin_v)
    pltpu.sync_copy(xt.at[f], xlin_v)

    def lb(s0, c):
        base = s0 * (U * L)
        for u in range(U):
            sl = pl.ds(base + u * L, L)
            idx = xlin_v[sl]
            vals = plsc.load_gather(lin_v, [idx])
            acc_v[sl] = vals * lin_splat
        return c
    lax.fori_loop(0, NB // U, lb, 0)

    for p in range(11):
        t1, t2, xi, xj, _ = bufs[p % 2]
        nxt = start(p + 1) if p + 1 < 11 else None
        for dsc in descs:
            dsc.wait()
        descs = nxt
        _, _, _, _, ro1, ro2, scale = _pair_params(wid, p)
        roff1 = jnp.full((L,), ro1, jnp.int32)
        roff2 = jnp.full((L,), ro2, jnp.int32)
        scale_splat = jnp.full((L,), scale, jnp.float32)

        def pb(s0, c):
            base = s0 * (U * L)
            for u in range(U):
                sl = pl.ds(base + u * L, L)
                ii = xi[sl] * D
                jj = xj[sl] * D
                tot = None
                for d in range(D):
                    fi = ii + d
                    fj = jj + d
                    e1 = plsc.load_gather(t1, [(fi >> 7) + roff1, fi & 127])
                    e2 = plsc.load_gather(t2, [(fj >> 7) + roff2, fj & 127])
                    prod = e1 * e2
                    tot = prod if tot is None else tot + prod
                acc_v[sl] = acc_v[sl] + tot * scale_splat
            return c
        lax.fori_loop(0, NB // U, pb, 0)

    pltpu.sync_copy(acc_v, part.at[wid])


def _k2_body(part, bias16, out, buf_v, bias_v, out_v):
    cid = lax.axis_index("c")
    sid = lax.axis_index("s")
    wid = sid * NC + cid
    bw = B // NW
    cbase = wid * bw

    pltpu.sync_copy(part.at[:, pl.ds(cbase, bw)], buf_v)
    pltpu.sync_copy(bias16, bias_v)
    bv = bias_v[...]

    for v in range(bw // L):
        acc = jnp.zeros((L,), jnp.float32)
        for r in range(NW):
            acc = acc + buf_v[r, pl.ds(v * L, L)]
        z = acc + bv
        res = 1.0 / (1.0 + jnp.exp(-z))
        out_v[pl.ds(v * L, L)] = res

    pltpu.sync_copy(out_v, out.at[pl.ds(cbase, bw)])


_mesh = plsc.VectorSubcoreMesh(core_axis_name="c", subcore_axis_name="s")

_cparams = pltpu.CompilerParams(needs_layout_passes=False,
                               use_tc_tiling_on_sc=True)

_k1 = pl.kernel(
    _k1_body,
    out_type=jax.ShapeDtypeStruct((NW, B), jnp.float32),
    mesh=_mesh,
    compiler_params=_cparams,
    scratch_types=[
        pltpu.VMEM((TW, 128), jnp.float32),
        pltpu.VMEM((TW, 128), jnp.float32),
        pltpu.VMEM((TW, 128), jnp.float32),
        pltpu.VMEM((TW, 128), jnp.float32),
        pltpu.VMEM((B,), jnp.int32),
        pltpu.VMEM((B,), jnp.int32),
        pltpu.VMEM((B,), jnp.int32),
        pltpu.VMEM((B,), jnp.int32),
        pltpu.VMEM((VF,), jnp.float32),
        pltpu.VMEM((B,), jnp.int32),
        pltpu.VMEM((B,), jnp.float32),
        pltpu.SemaphoreType.DMA,
        pltpu.SemaphoreType.DMA,
    ],
)

_k2 = pl.kernel(
    _k2_body,
    out_type=jax.ShapeDtypeStruct((B,), jnp.float32),
    mesh=_mesh,
    compiler_params=_cparams,
    scratch_types=[
        pltpu.VMEM((NW, B // NW), jnp.float32),
        pltpu.VMEM((L,), jnp.float32),
        pltpu.VMEM((B // NW,), jnp.float32),
    ],
)


@jax.jit
def kernel(x, W_lin, W_cross, bias):
    wcr = W_cross.reshape(F, F * VF * D // 128, 128)
    wlin = W_lin.reshape(F, VF)
    xt = x.T
    bias16 = jnp.broadcast_to(bias.astype(jnp.float32), (L,))
    part = _k1(wcr, wlin, xt)
    return _k2(part, bias16)
